# baseline (speedup 1.0000x reference)
import functools

import jax
import jax.numpy as jnp
from jax import lax
from jax.experimental import pallas as pl
from jax.experimental.pallas import tpu as pltpu
from jax.experimental.pallas import tpu_sc as plsc

BATCH = 4096
ENT = 100
DIM = 64
NC, NS = 2, 16
NW = NC * NS
BPW = BATCH // NW
IPW = BPW * ENT
OPW = 2 * IPW
NCHUNK = IPW // 16
SUB = 128
GR = 512
NSUB = GR // SUB
NG = OPW // GR


def _sc_body(x_hbm, tab_hbm, out_hbm, xv, comb, gbuf0, gbuf1,
             gs0, gs1, os0, os1):
    wid = lax.axis_index("s") * NC + lax.axis_index("c")

    pltpu.sync_copy(x_hbm.at[pl.ds(wid * IPW, IPW)], xv)

    def ibody(i, carry):
        base = i * 16
        v = xv[pl.ds(base, 16)]
        pos = base + lax.iota(jnp.int32, 16)
        neg = v < 0
        cidx = jnp.where(neg, 2, v + 2)
        mrow = jnp.where(neg, 0, 1).astype(jnp.int32)
        b = pos // ENT
        e = pos - b * ENT
        dv = b * (2 * ENT) + e
        plsc.store_scatter(comb, [dv], cidx)
        plsc.store_scatter(comb, [dv + ENT], mrow)
        return carry

    lax.fori_loop(0, NCHUNK, ibody, 0)

    out_base = wid * OPW
    bufs = ((gbuf0, gs0, os0), (gbuf1, gs1, os1))

    def gbody(t2, carry):
        for bi in range(2):
            buf, gsem, osem = bufs[bi]
            t = t2 * 2 + bi

            @pl.when(t2 > 0)
            def _wait_prev_out():
                pltpu.make_async_copy(buf, out_hbm.at[pl.ds(0, GR)],
                                      osem).wait()

            descs = []
            for s in range(NSUB):
                descs.append(pltpu.async_copy(
                    tab_hbm.at[comb.at[pl.ds(t * GR + s * SUB, SUB)]],
                    buf.at[pl.ds(s * SUB, SUB)], gsem))
            for d in descs:
                d.wait()
            pltpu.async_copy(buf, out_hbm.at[pl.ds(out_base + t * GR, GR)],
                             osem)
        return carry

    lax.fori_loop(0, NG // 2, gbody, 0)

    for buf, _, osem in bufs:
        pltpu.make_async_copy(buf, out_hbm.at[pl.ds(0, GR)], osem).wait()


_sc_call = functools.partial(
    pl.kernel,
    out_type=jax.ShapeDtypeStruct((BATCH * 2 * ENT, DIM), jnp.float32),
    mesh=plsc.VectorSubcoreMesh(core_axis_name="c", subcore_axis_name="s",
                                num_cores=NC, num_subcores=NS),
    scratch_types=[
        pltpu.VMEM((IPW,), jnp.int32),
        pltpu.VMEM((OPW,), jnp.int32),
        pltpu.VMEM((GR, DIM), jnp.float32),
        pltpu.VMEM((GR, DIM), jnp.float32),
        pltpu.SemaphoreType.DMA,
        pltpu.SemaphoreType.DMA,
        pltpu.SemaphoreType.DMA,
        pltpu.SemaphoreType.DMA,
    ],
    compiler_params=pltpu.CompilerParams(use_tc_tiling_on_sc=False,
                                         needs_layout_passes=False),
)(_sc_body)


def kernel(x, entity_vectors):
    mask_rows = jnp.concatenate(
        [jnp.zeros((1, DIM), jnp.float32), jnp.ones((1, DIM), jnp.float32)],
        axis=0)
    aug = jnp.concatenate([mask_rows, entity_vectors], axis=0)
    out = _sc_call(x.reshape(-1), aug)
    return out.reshape(BATCH, 2, ENT, DIM)


# speedup vs baseline: 8.2433x; 8.2433x over previous
import functools

import jax
import jax.numpy as jnp
from jax import lax
from jax.experimental import pallas as pl
from jax.experimental.pallas import tpu as pltpu
from jax.experimental.pallas import tpu_sc as plsc

BATCH = 4096
ENT = 100
DIM = 64
NC, NS = 2, 16
NW = NC * NS
BPW = BATCH // NW
IPW = BPW * ENT
OPW = 2 * IPW
NCHUNK = IPW // 16
SUB = 128
GR = 512
NSUB = GR // SUB
NG = OPW // GR
MSPREAD = 256


def _sc_body(x_hbm, tab_hbm, out_hbm, xv, comb, gbuf0, gbuf1,
             gs0, gs1, os0, os1):
    wid = lax.axis_index("s") * NC + lax.axis_index("c")

    pltpu.sync_copy(x_hbm.at[pl.ds(wid * IPW, IPW)], xv)

    def ibody(i, carry):
        base = i * 16
        v = xv[pl.ds(base, 16)]
        pos = base + lax.iota(jnp.int32, 16)
        neg = v < 0
        cidx = jnp.where(neg, 2 * MSPREAD, v + 2 * MSPREAD)
        mrow = jnp.where(neg, 0, MSPREAD) + (pos & (MSPREAD - 1))
        b = pos // ENT
        e = pos - b * ENT
        dv = b * (2 * ENT) + e
        plsc.store_scatter(comb, [dv], cidx)
        plsc.store_scatter(comb, [dv + ENT], mrow)
        return carry

    lax.fori_loop(0, NCHUNK, ibody, 0)

    out_base = wid * OPW
    bufs = ((gbuf0, gs0, os0), (gbuf1, gs1, os1))

    def gbody(t2, carry):
        for bi in range(2):
            buf, gsem, osem = bufs[bi]
            t = t2 * 2 + bi

            @pl.when(t2 > 0)
            def _wait_prev_out():
                pltpu.make_async_copy(buf, out_hbm.at[pl.ds(0, GR)],
                                      osem).wait()

            descs = []
            for s in range(NSUB):
                descs.append(pltpu.async_copy(
                    tab_hbm.at[comb.at[pl.ds(t * GR + s * SUB, SUB)]],
                    buf.at[pl.ds(s * SUB, SUB)], gsem))
            for d in descs:
                d.wait()
            pltpu.async_copy(buf, out_hbm.at[pl.ds(out_base + t * GR, GR)],
                             osem)
        return carry

    lax.fori_loop(0, NG // 2, gbody, 0)

    for buf, _, osem in bufs:
        pltpu.make_async_copy(buf, out_hbm.at[pl.ds(0, GR)], osem).wait()


_sc_call = functools.partial(
    pl.kernel,
    out_type=jax.ShapeDtypeStruct((BATCH * 2 * ENT, DIM), jnp.float32),
    mesh=plsc.VectorSubcoreMesh(core_axis_name="c", subcore_axis_name="s",
                                num_cores=NC, num_subcores=NS),
    scratch_types=[
        pltpu.VMEM((IPW,), jnp.int32),
        pltpu.VMEM((OPW,), jnp.int32),
        pltpu.VMEM((GR, DIM), jnp.float32),
        pltpu.VMEM((GR, DIM), jnp.float32),
        pltpu.SemaphoreType.DMA,
        pltpu.SemaphoreType.DMA,
        pltpu.SemaphoreType.DMA,
        pltpu.SemaphoreType.DMA,
    ],
    compiler_params=pltpu.CompilerParams(use_tc_tiling_on_sc=False,
                                         needs_layout_passes=False),
)(_sc_body)


def kernel(x, entity_vectors):
    mask_rows = jnp.concatenate(
        [jnp.zeros((MSPREAD, DIM), jnp.float32),
         jnp.ones((MSPREAD, DIM), jnp.float32)], axis=0)
    aug = jnp.concatenate([mask_rows, entity_vectors], axis=0)
    out = _sc_call(x.reshape(-1), aug)
    return out.reshape(BATCH, 2, ENT, DIM)


# speedup vs baseline: 11.4372x; 1.3875x over previous
import functools

import jax
import jax.numpy as jnp
from jax import lax
from jax.experimental import pallas as pl
from jax.experimental.pallas import tpu as pltpu
from jax.experimental.pallas import tpu_sc as plsc

BATCH = 4096
ENT = 100
DIM = 64
NC, NS = 2, 16
NW = NC * NS
BPW = BATCH // NW
IPW = BPW * ENT
OPW = 2 * IPW
NCHUNK = IPW // 16
GB = 2
GR = GB * 2 * ENT
NG = OPW // GR
EPAD = 112


def _sc_body(x_hbm, tab_hbm, out_hbm, xv, comb, mval, gbuf0, gbuf1,
             gs0, gs1, os0, os1):
    wid = lax.axis_index("s") * NC + lax.axis_index("c")

    pltpu.sync_copy(x_hbm.at[pl.ds(wid * IPW, IPW)], xv)

    def ibody(i, carry):
        base = i * 16
        v = xv[pl.ds(base, 16)]
        pos = base + lax.iota(jnp.int32, 16)
        neg = v < 0
        cidx = jnp.where(neg, 0, v)
        b = pos // ENT
        e = pos - b * ENT
        dv = b * (2 * ENT) + e
        plsc.store_scatter(comb, [dv], cidx)
        plsc.store_scatter(mval, [b * EPAD + e],
                           jnp.where(neg, 0.0, 1.0).astype(jnp.float32))
        return carry

    lax.fori_loop(0, NCHUNK, ibody, 0)

    out_base = wid * OPW
    bufs = ((gbuf0, gs0, os0), (gbuf1, gs1, os1))

    def gbody(t2, carry):
        for bi in range(2):
            buf, gsem, osem = bufs[bi]
            t = t2 * 2 + bi

            @pl.when(t2 > 0)
            def _wait_prev_out():
                pltpu.make_async_copy(buf, out_hbm.at[pl.ds(0, GR)],
                                      osem).wait()

            descs = []
            for c in range(GB):
                descs.append(pltpu.async_copy(
                    tab_hbm.at[comb.at[pl.ds(t * GR + c * 2 * ENT, ENT)]],
                    buf.at[pl.ds(c * 2 * ENT, ENT)], gsem))
            for c in range(GB):
                mbase = (t * GB + c) * EPAD
                for j in range(EPAD // 16):
                    m16 = mval[pl.ds(mbase + j * 16, 16)]
                    for l in range(16):
                        e = j * 16 + l
                        if e >= ENT:
                            continue
                        splat = jnp.full((16,), m16[l], jnp.float32)
                        row = c * 2 * ENT + ENT + e
                        for k in range(DIM // 16):
                            buf[row, pl.ds(k * 16, 16)] = splat
            for d in descs:
                d.wait()
            pltpu.async_copy(buf, out_hbm.at[pl.ds(out_base + t * GR, GR)],
                             osem)
        return carry

    lax.fori_loop(0, NG // 2, gbody, 0)

    for buf, _, osem in bufs:
        pltpu.make_async_copy(buf, out_hbm.at[pl.ds(0, GR)], osem).wait()


_sc_call = functools.partial(
    pl.kernel,
    out_type=jax.ShapeDtypeStruct((BATCH * 2 * ENT, DIM), jnp.float32),
    mesh=plsc.VectorSubcoreMesh(core_axis_name="c", subcore_axis_name="s",
                                num_cores=NC, num_subcores=NS),
    scratch_types=[
        pltpu.VMEM((IPW,), jnp.int32),
        pltpu.VMEM((OPW,), jnp.int32),
        pltpu.VMEM((BPW * EPAD,), jnp.float32),
        pltpu.VMEM((GR, DIM), jnp.float32),
        pltpu.VMEM((GR, DIM), jnp.float32),
        pltpu.SemaphoreType.DMA,
        pltpu.SemaphoreType.DMA,
        pltpu.SemaphoreType.DMA,
        pltpu.SemaphoreType.DMA,
    ],
    compiler_params=pltpu.CompilerParams(use_tc_tiling_on_sc=False,
                                         needs_layout_passes=False),
)(_sc_body)


def kernel(x, entity_vectors):
    out = _sc_call(x.reshape(-1), entity_vectors)
    return out.reshape(BATCH, 2, ENT, DIM)
